# baseline (speedup 1.0000x reference)
import functools
import jax
import jax.numpy as jnp
from jax import lax
from jax.experimental import pallas as pl
from jax.experimental.pallas import tpu as pltpu
from jax.experimental.pallas import tpu_sc as plsc

CONTEXT = 2048
EMBED = 128
NBATCH = 4
NC, NS, L = 2, 16, 16
NW = NC * NS
ROWS = NBATCH * CONTEXT
R_PER_W = ROWS // NW
N_CHUNKS = 8
CHUNK = R_PER_W // N_CHUNKS


def _sc_embed(token_ids_2d, token_table, pos_table):
    mesh = plsc.VectorSubcoreMesh(core_axis_name="c", subcore_axis_name="s")

    @functools.partial(
        pl.kernel,
        out_type=jax.ShapeDtypeStruct((ROWS, EMBED), jnp.float32),
        mesh=mesh,
        scratch_types=[
            pltpu.VMEM((N_CHUNKS, CHUNK), jnp.int32),
            pltpu.VMEM((R_PER_W, EMBED), jnp.float32),
            pltpu.SemaphoreType.DMA,
        ]
        + [pltpu.SemaphoreType.DMA] * N_CHUNKS
        + [pltpu.SemaphoreType.DMA] * N_CHUNKS,
    )
    def body(ids_hbm, table_hbm, pos_hbm, out_hbm, idx_v, rows_v, sem_io, *sems):
        sem_pos = sems[:N_CHUNKS]
        sem_g = sems[N_CHUNKS:]
        wid = lax.axis_index("s") * NC + lax.axis_index("c")
        base = wid * R_PER_W
        pos_base = lax.rem(wid, CONTEXT // R_PER_W) * R_PER_W

        idx_cp = pltpu.async_copy(
            ids_hbm.at[pl.ds(wid * N_CHUNKS, N_CHUNKS)], idx_v, sem_io
        )
        pos_cps = [
            pltpu.async_copy(
                pos_hbm.at[pl.ds(pos_base + j * CHUNK, CHUNK)],
                rows_v.at[pl.ds(j * CHUNK, CHUNK)],
                sem_pos[j],
            )
            for j in range(N_CHUNKS)
        ]
        idx_cp.wait()

        gathers = []
        for j in range(N_CHUNKS):
            pos_cps[j].wait()
            gathers.append(
                pltpu.async_copy(
                    table_hbm.at[idx_v.at[j]],
                    rows_v.at[pl.ds(j * CHUNK, CHUNK)],
                    sem_g[j],
                    add=True,
                )
            )
        writes = []
        for j in range(N_CHUNKS):
            gathers[j].wait()
            writes.append(
                pltpu.async_copy(
                    rows_v.at[pl.ds(j * CHUNK, CHUNK)],
                    out_hbm.at[pl.ds(base + j * CHUNK, CHUNK)],
                    sem_io,
                )
            )
        for w in writes:
            w.wait()

    return body(token_ids_2d, token_table, pos_table)


def kernel(token_ids, token_table, pos_table):
    ids_flat = token_ids.astype(jnp.int32).reshape(ROWS // CHUNK, CHUNK)
    out = _sc_embed(ids_flat, token_table, pos_table)
    return out.reshape(NBATCH, CONTEXT, EMBED)


# speedup vs baseline: 1.0151x; 1.0151x over previous
import functools
import jax
import jax.numpy as jnp
from jax import lax
from jax.experimental import pallas as pl
from jax.experimental.pallas import tpu as pltpu
from jax.experimental.pallas import tpu_sc as plsc

CONTEXT = 2048
EMBED = 128
NBATCH = 4
NC, NS, L = 2, 16, 16
NW = NC * NS
ROWS = NBATCH * CONTEXT
R_PER_W = ROWS // NW
N_CHUNKS = 4
CHUNK = R_PER_W // N_CHUNKS
P_SLICES = CONTEXT // R_PER_W
P_PER_SC = P_SLICES // NC


def _sc_embed(token_ids_2d, token_table, pos_table):
    mesh = plsc.VectorSubcoreMesh(core_axis_name="c", subcore_axis_name="s")

    @functools.partial(
        pl.kernel,
        out_type=jax.ShapeDtypeStruct((ROWS, EMBED), jnp.float32),
        mesh=mesh,
        scratch_types=[
            pltpu.VMEM((N_CHUNKS, CHUNK), jnp.int32),
            pltpu.VMEM((R_PER_W, EMBED), jnp.float32),
            pltpu.VMEM_SHARED((P_PER_SC * R_PER_W, EMBED), jnp.float32),
            pltpu.SemaphoreType.DMA,
        ]
        + [pltpu.SemaphoreType.DMA] * N_CHUNKS
        + [pltpu.SemaphoreType.DMA] * N_CHUNKS,
    )
    def body(
        ids_hbm, table_hbm, pos_hbm, out_hbm, idx_v, rows_v, pos_sh, sem_io, *sems
    ):
        sem_pos = sems[:N_CHUNKS]
        sem_g = sems[N_CHUNKS:]
        c = lax.axis_index("c")
        s = lax.axis_index("s")
        b = s // P_PER_SC
        p_local = lax.rem(s, P_PER_SC)
        p = P_PER_SC * c + p_local
        chunkid = b * P_SLICES + p
        base = chunkid * R_PER_W

        idx_cp = pltpu.async_copy(
            ids_hbm.at[pl.ds(chunkid * N_CHUNKS, N_CHUNKS)], idx_v, sem_io
        )
        @pl.when(s < P_PER_SC)
        def _():
            pltpu.sync_copy(
                pos_hbm.at[pl.ds((P_PER_SC * c + s) * R_PER_W, R_PER_W)],
                pos_sh.at[pl.ds(s * R_PER_W, R_PER_W)],
            )

        plsc.subcore_barrier()

        pos_cps = [
            pltpu.async_copy(
                pos_sh.at[pl.ds(p_local * R_PER_W + j * CHUNK, CHUNK)],
                rows_v.at[pl.ds(j * CHUNK, CHUNK)],
                sem_pos[j],
            )
            for j in range(N_CHUNKS)
        ]
        idx_cp.wait()

        gathers = []
        for j in range(N_CHUNKS):
            pos_cps[j].wait()
            gathers.append(
                pltpu.async_copy(
                    table_hbm.at[idx_v.at[j]],
                    rows_v.at[pl.ds(j * CHUNK, CHUNK)],
                    sem_g[j],
                    add=True,
                )
            )
        writes = []
        for j in range(N_CHUNKS):
            gathers[j].wait()
            writes.append(
                pltpu.async_copy(
                    rows_v.at[pl.ds(j * CHUNK, CHUNK)],
                    out_hbm.at[pl.ds(base + j * CHUNK, CHUNK)],
                    sem_io,
                )
            )
        for w in writes:
            w.wait()

    return body(token_ids_2d, token_table, pos_table)


def kernel(token_ids, token_table, pos_table):
    ids_flat = token_ids.astype(jnp.int32).reshape(ROWS // CHUNK, CHUNK)
    out = _sc_embed(ids_flat, token_table, pos_table)
    return out.reshape(NBATCH, CONTEXT, EMBED)
